# baseline (speedup 1.0000x reference)
import functools

import jax
import jax.numpy as jnp
from jax import lax
from jax.experimental import pallas as pl
from jax.experimental.pallas import tpu as pltpu
from jax.experimental.pallas import tpu_sc as plsc

NUM_CLASSES = 1000000
FEAT_DIM = 64
BATCH = 16384

NC = 2
NS = 16
LANES = 16
NW = NC * NS
BLK = 512
BLK_SHIFT = 9
NBLK = (NUM_CLASSES + BLK - 1) // BLK
BLK_PER_W = (NBLK + NW - 1) // NW
CLS_PAD = ((NUM_CLASSES + 127) // 128) * 128
MAX_START = CLS_PAD - BLK
NVREG = BATCH // LANES
FLUSH = 64
SCAT_ROWS = BATCH + FLUSH


def _sc_body(cent_hbm, tgt_hbm, scat_hbm,
             tgt_v, tm_un, bm_un, bufs, outbuf, bm_chunk,
             ssem, scsem, tsem):
    wid = lax.axis_index("s") * NC + lax.axis_index("c")
    lo = wid * BLK_PER_W
    nblk = jnp.minimum(NBLK - lo, BLK_PER_W)

    lane = lax.iota(jnp.int32, LANES)
    dvecs = [q * LANES + lane for q in range(FEAT_DIM // LANES)]

    pltpu.async_copy(tgt_hbm, tgt_v, tsem).wait()

    def extract(j, cnt):
        tv = tgt_v[pl.ds(j * LANES, LANES)]
        civ = lax.shift_right_logical(tv, BLK_SHIFT)
        inm = (civ >= lo) & (civ < lo + nblk)
        cs = plsc.cumsum(inm.astype(jnp.int32))
        pos = cnt + cs - 1
        plsc.store_scatter(tm_un, [pos], tv, mask=inm)
        plsc.store_scatter(bm_un, [pos], j * LANES + lane, mask=inm)
        return cnt + cs[15]

    nm = lax.fori_loop(0, NVREG, extract, jnp.int32(0))
    tm_un[pl.ds(nm, LANES)] = jnp.full((LANES,), 0x40000000, jnp.int32)
    nmv = (nm + LANES - 1) // LANES

    def reset_chunk():
        for r in range(FLUSH // LANES):
            bm_chunk[pl.ds(r * LANES, LANES)] = BATCH + r * LANES + lane

    reset_chunk()

    zeros = jnp.zeros((LANES,), jnp.float32)
    for r in range(FLUSH):
        for q in range(FEAT_DIM // LANES):
            outbuf[r, pl.ds(FEAT_DIM + q * LANES, LANES)] = zeros

    def blk_start(g):
        return pl.multiple_of(jnp.minimum((lo + g) * BLK, MAX_START), 128)

    def blk_src(g):
        return cent_hbm.at[:, pl.ds(blk_start(g), BLK)]

    pltpu.async_copy(blk_src(0), bufs.at[0], ssem)

    def stream(g, cnt_rows):
        @pl.when(g + 1 < nblk)
        def _():
            pltpu.async_copy(blk_src(g + 1), bufs.at[(g + 1) & 1], ssem)

        pltpu.make_async_copy(blk_src(0), bufs.at[g & 1], ssem).wait()
        gbuf = jnp.full((LANES,), g & 1, jnp.int32)
        start_cls = blk_start(g)

        def scan_vreg(k, cnt_rows):
            tmv = tm_un[pl.ds(k * LANES, LANES)]
            civ = lax.shift_right_logical(tmv, BLK_SHIFT)
            m = civ == lo + g

            def cond(carry):
                m, _ = carry
                return plsc.all_reduce_population_count(m)[0] > 0

            def body(carry):
                m, cnt_rows = carry
                lf = plsc.all_reduce_ffs(m)
                idx = k * LANES + lf
                tmval = plsc.load_gather(tm_un, [idx])
                bmval = plsc.load_gather(bm_un, [idx])
                l_spl = tmval - start_cls
                row = cnt_rows & (FLUSH - 1)
                for q in range(FEAT_DIM // LANES):
                    cv = plsc.load_gather(bufs, [gbuf, dvecs[q], l_spl])
                    outbuf[row, pl.ds(q * LANES, LANES)] = cv
                plsc.store_scatter(bm_chunk, [jnp.full((LANES,), row, jnp.int32)],
                                   bmval, mask=lane == 0)
                cnt_rows = cnt_rows + 1

                @pl.when((cnt_rows & (FLUSH - 1)) == 0)
                def _():
                    pltpu.async_copy(outbuf, scat_hbm.at[bm_chunk], scsem).wait()
                    reset_chunk()

                return m & (lane != lf), cnt_rows

            _, cnt_rows = lax.while_loop(cond, body, (m, cnt_rows))
            return cnt_rows

        return lax.fori_loop(0, nmv, scan_vreg, cnt_rows)

    cnt_rows = lax.fori_loop(0, nblk, stream, jnp.int32(0))

    @pl.when((cnt_rows & (FLUSH - 1)) != 0)
    def _():
        pltpu.async_copy(outbuf, scat_hbm.at[bm_chunk], scsem).wait()


_sc_gather = functools.partial(
    pl.kernel,
    out_type=jax.ShapeDtypeStruct((SCAT_ROWS, 128), jnp.float32),
    mesh=plsc.VectorSubcoreMesh(core_axis_name="c", subcore_axis_name="s"),
    compiler_params=pltpu.CompilerParams(needs_layout_passes=False),
    scratch_types=[
        pltpu.VMEM((BATCH,), jnp.int32),
        pltpu.VMEM((BATCH + LANES,), jnp.int32),
        pltpu.VMEM((BATCH + LANES,), jnp.int32),
        pltpu.VMEM((2, FEAT_DIM, BLK), jnp.float32),
        pltpu.VMEM((FLUSH, 128), jnp.float32),
        pltpu.VMEM((FLUSH,), jnp.int32),
        pltpu.SemaphoreType.DMA,
        pltpu.SemaphoreType.DMA,
        pltpu.SemaphoreType.DMA,
    ],
)(_sc_body)


GRID = 32
FBLK = BATCH // GRID


def _tc_body(ft_ref, sc_ref, o_ref, acc_ref, s_ref):
    i = pl.program_id(0)

    @pl.when(i == 0)
    def _():
        acc_ref[...] = jnp.zeros_like(acc_ref)
        s_ref[0] = 0.0
        s_ref[1] = 0.0

    f = ft_ref[...]
    c = sc_ref[...]
    acc_ref[...] += lax.dot_general(
        f, c, (((1,), (0,)), ((), ())),
        preferred_element_type=jnp.float32)
    s_ref[0] += jnp.sum(f * f)
    s_ref[1] += jnp.sum(c[:, :FEAT_DIM] * c[:, :FEAT_DIM])

    @pl.when(i == GRID - 1)
    def _():
        r = lax.broadcasted_iota(jnp.int32, (FEAT_DIM, 128), 0)
        col = lax.broadcasted_iota(jnp.int32, (FEAT_DIM, 128), 1)
        tr = jnp.sum(jnp.where(r == col, acc_ref[...], 0.0))
        o_ref[0, 0] = (s_ref[0] + s_ref[1] - 2.0 * tr) * (1.0 / BATCH)


_tc_finish = pl.pallas_call(
    _tc_body,
    grid=(GRID,),
    in_specs=[
        pl.BlockSpec((FEAT_DIM, FBLK), lambda i: (0, i)),
        pl.BlockSpec((FBLK, 128), lambda i: (i, 0)),
    ],
    out_specs=pl.BlockSpec(memory_space=pltpu.SMEM),
    out_shape=jax.ShapeDtypeStruct((1, 1), jnp.float32),
    scratch_shapes=[
        pltpu.VMEM((FEAT_DIM, 128), jnp.float32),
        pltpu.SMEM((2,), jnp.float32),
    ],
)


def kernel(features, targets, centers):
    tgt = targets.astype(jnp.int32)
    scat = _sc_gather(centers.T, tgt)
    return _tc_finish(features.T, scat)[0, 0]


# speedup vs baseline: 1.0070x; 1.0070x over previous
import functools

import jax
import jax.numpy as jnp
from jax import lax
from jax.experimental import pallas as pl
from jax.experimental.pallas import tpu as pltpu
from jax.experimental.pallas import tpu_sc as plsc

NUM_CLASSES = 1000000
FEAT_DIM = 64
BATCH = 16384

NC = 2
NS = 16
LANES = 16
NW = NC * NS
BLK = 512
BLK_SHIFT = 9
NBLK = (NUM_CLASSES + BLK - 1) // BLK
BLK_PER_W = (NBLK + NW - 1) // NW
CLS_PAD = ((NUM_CLASSES + 127) // 128) * 128
MAX_START = CLS_PAD - BLK
NVREG = BATCH // LANES
FLUSH = 64
SCAT_ROWS = BATCH + FLUSH


def _sc_body(cent_hbm, tgt_hbm, scat_hbm,
             tgt_v, tm_un, bm_un, bufs, outbuf, bm_chunk,
             ssem, scsem, tsem):
    wid = lax.axis_index("s") * NC + lax.axis_index("c")
    lo = wid * BLK_PER_W
    nblk = jnp.minimum(NBLK - lo, BLK_PER_W)

    lane = lax.iota(jnp.int32, LANES)
    dvecs = [q * LANES + lane for q in range(FEAT_DIM // LANES)]

    def blk_start(g):
        return pl.multiple_of(jnp.minimum((lo + g) * BLK, MAX_START), 128)

    def issue_blk(g, p):
        s = blk_start(g)
        pltpu.async_copy(cent_hbm.at[pl.ds(0, 32), pl.ds(s, BLK)],
                         bufs.at[p].at[pl.ds(0, 32)], ssem)
        pltpu.async_copy(cent_hbm.at[pl.ds(32, 32), pl.ds(s, BLK)],
                         bufs.at[p].at[pl.ds(32, 32)], ssem)

    def wait_blk(p):
        pltpu.make_async_copy(cent_hbm.at[pl.ds(0, 64), pl.ds(0, BLK)],
                              bufs.at[p], ssem).wait()

    tcopy = pltpu.async_copy(tgt_hbm, tgt_v, tsem)
    issue_blk(0, 0)
    issue_blk(1, 1)
    tcopy.wait()

    def extract(j, cnt):
        tv = tgt_v[pl.ds(j * LANES, LANES)]
        civ = lax.shift_right_logical(tv, BLK_SHIFT)
        inm = (civ >= lo) & (civ < lo + nblk)
        cs = plsc.cumsum(inm.astype(jnp.int32))
        pos = cnt + cs - 1
        plsc.store_scatter(tm_un, [pos], tv, mask=inm)
        plsc.store_scatter(bm_un, [pos], j * LANES + lane, mask=inm)
        return cnt + cs[15]

    nm = lax.fori_loop(0, NVREG, extract, jnp.int32(0))
    tm_un[pl.ds(nm, LANES)] = jnp.full((LANES,), 0x40000000, jnp.int32)
    nmv = (nm + LANES - 1) // LANES

    def reset_chunk():
        for r in range(FLUSH // LANES):
            bm_chunk[pl.ds(r * LANES, LANES)] = BATCH + r * LANES + lane

    reset_chunk()

    zeros = jnp.zeros((LANES,), jnp.float32)
    for r in range(FLUSH):
        for q in range(FEAT_DIM // LANES):
            outbuf[r, pl.ds(FEAT_DIM + q * LANES, LANES)] = zeros

    def stream(g, cnt_rows):
        wait_blk(g & 1)

        gbuf = jnp.full((LANES,), g & 1, jnp.int32)
        start_cls = blk_start(g)

        def scan_vreg(k, cnt_rows):
            tmv = tm_un[pl.ds(k * LANES, LANES)]
            civ = lax.shift_right_logical(tmv, BLK_SHIFT)
            m = civ == lo + g

            def cond(carry):
                m, _ = carry
                return plsc.all_reduce_population_count(m)[0] > 0

            def body(carry):
                m, cnt_rows = carry
                lf = plsc.all_reduce_ffs(m)
                idx = k * LANES + lf
                tmval = plsc.load_gather(tm_un, [idx])
                bmval = plsc.load_gather(bm_un, [idx])
                l_spl = tmval - start_cls
                row = cnt_rows & (FLUSH - 1)
                for q in range(FEAT_DIM // LANES):
                    cv = plsc.load_gather(bufs, [gbuf, dvecs[q], l_spl])
                    outbuf[row, pl.ds(q * LANES, LANES)] = cv
                plsc.store_scatter(bm_chunk, [jnp.full((LANES,), row, jnp.int32)],
                                   bmval, mask=lane == 0)
                cnt_rows = cnt_rows + 1

                @pl.when((cnt_rows & (FLUSH - 1)) == 0)
                def _():
                    pltpu.async_copy(outbuf, scat_hbm.at[bm_chunk], scsem).wait()
                    reset_chunk()

                return m & (lane != lf), cnt_rows

            _, cnt_rows = lax.while_loop(cond, body, (m, cnt_rows))
            return cnt_rows

        cnt_rows = lax.fori_loop(0, nmv, scan_vreg, cnt_rows)

        @pl.when(g + 2 < nblk)
        def _():
            issue_blk(g + 2, g & 1)

        return cnt_rows

    cnt_rows = lax.fori_loop(0, nblk, stream, jnp.int32(0))

    @pl.when((cnt_rows & (FLUSH - 1)) != 0)
    def _():
        pltpu.async_copy(outbuf, scat_hbm.at[bm_chunk], scsem).wait()


_sc_gather = functools.partial(
    pl.kernel,
    out_type=jax.ShapeDtypeStruct((SCAT_ROWS, 128), jnp.float32),
    mesh=plsc.VectorSubcoreMesh(core_axis_name="c", subcore_axis_name="s"),
    compiler_params=pltpu.CompilerParams(needs_layout_passes=False),
    scratch_types=[
        pltpu.VMEM((BATCH,), jnp.int32),
        pltpu.VMEM((BATCH + LANES,), jnp.int32),
        pltpu.VMEM((BATCH + LANES,), jnp.int32),
        pltpu.VMEM((2, FEAT_DIM, BLK), jnp.float32),
        pltpu.VMEM((FLUSH, 128), jnp.float32),
        pltpu.VMEM((FLUSH,), jnp.int32),
        pltpu.SemaphoreType.DMA,
        pltpu.SemaphoreType.DMA,
        pltpu.SemaphoreType.DMA,
    ],
)(_sc_body)


GRID = 32
FBLK = BATCH // GRID


def _tc_body(ft_ref, sc_ref, o_ref, acc_ref, s_ref):
    i = pl.program_id(0)

    @pl.when(i == 0)
    def _():
        acc_ref[...] = jnp.zeros_like(acc_ref)
        s_ref[0] = 0.0
        s_ref[1] = 0.0

    f = ft_ref[...]
    c = sc_ref[...]
    acc_ref[...] += lax.dot_general(
        f, c, (((1,), (0,)), ((), ())),
        preferred_element_type=jnp.float32)
    s_ref[0] += jnp.sum(f * f)
    s_ref[1] += jnp.sum(c[:, :FEAT_DIM] * c[:, :FEAT_DIM])

    @pl.when(i == GRID - 1)
    def _():
        r = lax.broadcasted_iota(jnp.int32, (FEAT_DIM, 128), 0)
        col = lax.broadcasted_iota(jnp.int32, (FEAT_DIM, 128), 1)
        tr = jnp.sum(jnp.where(r == col, acc_ref[...], 0.0))
        o_ref[0, 0] = (s_ref[0] + s_ref[1] - 2.0 * tr) * (1.0 / BATCH)


_tc_finish = pl.pallas_call(
    _tc_body,
    grid=(GRID,),
    in_specs=[
        pl.BlockSpec((FEAT_DIM, FBLK), lambda i: (0, i)),
        pl.BlockSpec((FBLK, 128), lambda i: (i, 0)),
    ],
    out_specs=pl.BlockSpec(memory_space=pltpu.SMEM),
    out_shape=jax.ShapeDtypeStruct((1, 1), jnp.float32),
    scratch_shapes=[
        pltpu.VMEM((FEAT_DIM, 128), jnp.float32),
        pltpu.SMEM((2,), jnp.float32),
    ],
)


def kernel(features, targets, centers):
    tgt = targets.astype(jnp.int32)
    scat = _sc_gather(centers.T, tgt)
    return _tc_finish(features.T, scat)[0, 0]


# speedup vs baseline: 1.1002x; 1.0925x over previous
import functools

import jax
import jax.numpy as jnp
from jax import lax
from jax.experimental import pallas as pl
from jax.experimental.pallas import tpu as pltpu
from jax.experimental.pallas import tpu_sc as plsc

NUM_CLASSES = 1000000
FEAT_DIM = 64
BATCH = 16384

NC = 2
NS = 16
LANES = 16
NW = NC * NS
BLK = 512
BLK_SHIFT = 9
NBLK = (NUM_CLASSES + BLK - 1) // BLK
BLK_PER_W = (NBLK + NW - 1) // NW
CLS_PAD = ((NUM_CLASSES + 127) // 128) * 128
MAX_START = CLS_PAD - BLK
NVREG = BATCH // LANES
FLUSH = 64
SCAT_ROWS = BATCH + FLUSH


def _sc_body(cent_hbm, tgt_hbm, scat_hbm,
             tgt_v, tm_un, bm_un, bufs, outbuf, bm_chunk,
             ssem, scsem, tsem):
    wid = lax.axis_index("s") * NC + lax.axis_index("c")
    lo = wid * BLK_PER_W
    nblk = jnp.minimum(NBLK - lo, BLK_PER_W)

    lane = lax.iota(jnp.int32, LANES)
    dvecs = [q * LANES + lane for q in range(FEAT_DIM // LANES)]

    def blk_start(g):
        return pl.multiple_of(jnp.minimum((lo + g) * BLK, MAX_START), 128)

    def issue_blk(g, p):
        s = blk_start(g)
        pltpu.async_copy(cent_hbm.at[pl.ds(0, 32), pl.ds(s, BLK)],
                         bufs.at[p].at[pl.ds(0, 32)], ssem)
        pltpu.async_copy(cent_hbm.at[pl.ds(32, 32), pl.ds(s, BLK)],
                         bufs.at[p].at[pl.ds(32, 32)], ssem)

    def wait_blk(p):
        pltpu.make_async_copy(cent_hbm.at[pl.ds(0, 64), pl.ds(0, BLK)],
                              bufs.at[p], ssem).wait()

    tcopy = pltpu.async_copy(tgt_hbm, tgt_v, tsem)
    issue_blk(0, 0)
    issue_blk(1, 1)
    tcopy.wait()

    def extract(j, cnt):
        tv = tgt_v[pl.ds(j * LANES, LANES)]
        civ = lax.shift_right_logical(tv, BLK_SHIFT)
        inm = (civ >= lo) & (civ < lo + nblk)
        cs = plsc.cumsum(inm.astype(jnp.int32))
        pos = cnt + cs - 1
        plsc.store_scatter(tm_un, [pos], tv, mask=inm)
        plsc.store_scatter(bm_un, [pos], j * LANES + lane, mask=inm)
        return cnt + cs[15]

    nm = lax.fori_loop(0, NVREG, extract, jnp.int32(0))
    tm_un[pl.ds(nm, LANES)] = jnp.full((LANES,), 0x40000000, jnp.int32)
    nmv = (nm + LANES - 1) // LANES

    def reset_chunk():
        for r in range(FLUSH // LANES):
            bm_chunk[pl.ds(r * LANES, LANES)] = BATCH + r * LANES + lane

    reset_chunk()

    zeros = jnp.zeros((LANES,), jnp.float32)
    for r in range(FLUSH):
        for q in range(FEAT_DIM // LANES):
            outbuf[r, pl.ds(FEAT_DIM + q * LANES, LANES)] = zeros

    def stream(g, cnt_rows):
        wait_blk(g & 1)

        gbuf = jnp.full((LANES,), g & 1, jnp.int32)
        start_cls = blk_start(g)

        def scan_vreg(k, cnt_rows):
            tmv = tm_un[pl.ds(k * LANES, LANES)]
            civ = lax.shift_right_logical(tmv, BLK_SHIFT)
            m = civ == lo + g

            def cond(carry):
                m, _ = carry
                return plsc.all_reduce_population_count(m)[0] > 0

            def body(carry):
                m, cnt_rows = carry
                lf = plsc.all_reduce_ffs(m)
                idx = k * LANES + lf
                tmval = plsc.load_gather(tm_un, [idx])
                bmval = plsc.load_gather(bm_un, [idx])
                l_spl = tmval - start_cls
                row = cnt_rows & (FLUSH - 1)
                for q in range(FEAT_DIM // LANES):
                    cv = plsc.load_gather(bufs, [gbuf, dvecs[q], l_spl])
                    outbuf[row, pl.ds(q * LANES, LANES)] = cv
                plsc.store_scatter(bm_chunk, [jnp.full((LANES,), row, jnp.int32)],
                                   bmval, mask=lane == 0)
                cnt_rows = cnt_rows + 1

                @pl.when((cnt_rows & (FLUSH - 1)) == 0)
                def _():
                    pltpu.async_copy(outbuf, scat_hbm.at[bm_chunk], scsem).wait()
                    reset_chunk()

                return m & (lane != lf), cnt_rows

            _, cnt_rows = lax.while_loop(cond, body, (m, cnt_rows))
            return cnt_rows

        cnt_rows = lax.fori_loop(0, nmv, scan_vreg, cnt_rows)

        @pl.when(g + 2 < nblk)
        def _():
            issue_blk(g + 2, g & 1)

        return cnt_rows

    cnt_rows = lax.fori_loop(0, nblk, stream, jnp.int32(0))

    @pl.when((cnt_rows & (FLUSH - 1)) != 0)
    def _():
        pltpu.async_copy(outbuf, scat_hbm.at[bm_chunk], scsem).wait()


_sc_gather = functools.partial(
    pl.kernel,
    out_type=jax.ShapeDtypeStruct((SCAT_ROWS, 128), jnp.float32),
    mesh=plsc.VectorSubcoreMesh(core_axis_name="c", subcore_axis_name="s"),
    compiler_params=pltpu.CompilerParams(needs_layout_passes=False),
    scratch_types=[
        pltpu.VMEM((BATCH,), jnp.int32),
        pltpu.VMEM((BATCH + LANES,), jnp.int32),
        pltpu.VMEM((BATCH + LANES,), jnp.int32),
        pltpu.VMEM((2, FEAT_DIM, BLK), jnp.float32),
        pltpu.VMEM((FLUSH, 128), jnp.float32),
        pltpu.VMEM((FLUSH,), jnp.int32),
        pltpu.SemaphoreType.DMA,
        pltpu.SemaphoreType.DMA,
        pltpu.SemaphoreType.DMA,
    ],
)(_sc_body)


GRID = 4
FBLK = BATCH // GRID


def _tc_body(ft_ref, sc_ref, o_ref, acc_ref, s_ref):
    i = pl.program_id(0)

    @pl.when(i == 0)
    def _():
        acc_ref[...] = jnp.zeros_like(acc_ref)
        s_ref[0] = 0.0
        s_ref[1] = 0.0

    f = ft_ref[...]
    c = sc_ref[...]
    acc_ref[...] += lax.dot_general(
        f, c, (((1,), (0,)), ((), ())),
        preferred_element_type=jnp.float32)
    s_ref[0] += jnp.sum(f * f)
    s_ref[1] += jnp.sum(c[:, :FEAT_DIM] * c[:, :FEAT_DIM])

    @pl.when(i == GRID - 1)
    def _():
        r = lax.broadcasted_iota(jnp.int32, (FEAT_DIM, 128), 0)
        col = lax.broadcasted_iota(jnp.int32, (FEAT_DIM, 128), 1)
        tr = jnp.sum(jnp.where(r == col, acc_ref[...], 0.0))
        o_ref[0, 0] = (s_ref[0] + s_ref[1] - 2.0 * tr) * (1.0 / BATCH)


_tc_finish = pl.pallas_call(
    _tc_body,
    grid=(GRID,),
    in_specs=[
        pl.BlockSpec((FEAT_DIM, FBLK), lambda i: (0, i)),
        pl.BlockSpec((FBLK, 128), lambda i: (i, 0)),
    ],
    out_specs=pl.BlockSpec(memory_space=pltpu.SMEM),
    out_shape=jax.ShapeDtypeStruct((1, 1), jnp.float32),
    scratch_shapes=[
        pltpu.VMEM((FEAT_DIM, 128), jnp.float32),
        pltpu.SMEM((2,), jnp.float32),
    ],
)


def kernel(features, targets, centers):
    tgt = targets.astype(jnp.int32)
    scat = _sc_gather(centers.T, tgt)
    return _tc_finish(features.T, scat)[0, 0]
